# baseline (speedup 1.0000x reference)
import jax
import jax.numpy as jnp
from jax import lax
from jax.experimental import pallas as pl
from jax.experimental.pallas import tpu as pltpu
from jax.experimental.pallas import tpu_sc as plsc

N = 10000
E = 320000
H = 128
NC = 2
NS = 16
NW = NC * NS
CH = 40
NCHUNK = E // (NW * CH)
NB = 5
GB = 10
NG = NCHUNK // GB
NBODY = GB // NB
SB = 3
SROWS = GB + NB
IPAD = 16
DCH = 40
DNCHUNK = E // (NW * DCH)
DUN = 5
DGB = 25
DNG = DNCHUNK // DGB
DNI = DGB // DUN
STRIPE = 624
TAIL0 = NS * STRIPE
TAILN = N - TAIL0
BLK = 1000
GRID = N // BLK

_MESH = plsc.VectorSubcoreMesh(
    core_axis_name="c", subcore_axis_name="s", num_cores=NC, num_subcores=NS
)


def _sc_agg_body(h_hbm, src_hbm, dst_hbm, zeros_hbm,
                 out_agg,
                 src_bk, dst_st, rows_v, acc,
                 g0, g1, g2, g3, g4, ssem):
    c = lax.axis_index("c")
    s = lax.axis_index("s")
    w = c * NS + s
    r0 = s * STRIPE
    pltpu.sync_copy(zeros_hbm.at[pl.ds(r0, STRIPE), :], acc.at[pl.ds(r0, STRIPE), :])

    @pl.when(s == NS - 1)
    def _():
        pltpu.sync_copy(zeros_hbm.at[pl.ds(TAIL0, TAILN), :],
                        acc.at[pl.ds(TAIL0, TAILN), :])

    plsc.subcore_barrier()

    rows = [rows_v.at[k] for k in range(NB)]
    gsem = [g0, g1, g2, g3, g4]

    def prime(k):
        pltpu.async_copy(h_hbm.at[src_bk.at[0, k, 0]], rows[k], gsem[k])

    def group(g, carry):
        p = lax.rem(g, SB)
        pn = lax.rem(g + 1, SB)
        base = w * NCHUNK + g * GB
        pltpu.sync_copy(dst_hbm.at[pl.ds(base, GB)], dst_st)
        pltpu.sync_copy(src_hbm.at[pl.ds(base + GB, SROWS)], src_bk.at[pn])

        def body(i, carry2):
            t0 = NB * i
            for k in range(NB):
                lc = t0 + k
                pltpu.make_async_copy(
                    h_hbm.at[src_bk.at[p, lc, 0]], rows[k], gsem[k]).wait()
                pltpu.async_copy(
                    rows[k], acc.at[dst_st.at[lc, 0]], ssem, add=True).wait()
                pltpu.async_copy(h_hbm.at[src_bk.at[p, lc + NB, 0]], rows[k],
                                 gsem[k])
            return carry2

        lax.fori_loop(0, NBODY, body, 0)
        return carry

    pltpu.sync_copy(src_hbm.at[pl.ds(w * NCHUNK, SROWS)], src_bk.at[0])
    for k in range(NB):
        prime(k)
    lax.fori_loop(0, NG, group, 0)
    for k in range(NB):
        pltpu.make_async_copy(h_hbm.at[pl.ds(0, CH)], rows[k], gsem[k]).wait()
    plsc.subcore_barrier()
    pltpu.sync_copy(acc.at[pl.ds(r0, STRIPE), :], out_agg.at[c, pl.ds(r0, STRIPE), :])

    @pl.when(s == NS - 1)
    def _():
        pltpu.sync_copy(acc.at[pl.ds(TAIL0, TAILN), :],
                        out_agg.at[c, pl.ds(TAIL0, TAILN), :])


def _sc_deg_body(dst_hbm, zeros_hbm, ones_hbm,
                 out_deg,
                 dst_c, ones_v, accd, ssem):
    c = lax.axis_index("c")
    s = lax.axis_index("s")
    w = c * NS + s
    r0 = s * STRIPE
    pltpu.sync_copy(zeros_hbm.at[pl.ds(r0, STRIPE), :], accd.at[pl.ds(r0, STRIPE), :])

    @pl.when(s == NS - 1)
    def _():
        pltpu.sync_copy(zeros_hbm.at[pl.ds(TAIL0, TAILN), :],
                        accd.at[pl.ds(TAIL0, TAILN), :])

    pltpu.sync_copy(ones_hbm, ones_v)
    plsc.subcore_barrier()

    def group(g, carry):
        base = w * DNCHUNK + g * DGB
        pltpu.sync_copy(dst_hbm.at[pl.ds(base, DGB)], dst_c)

        def step(k, carry2):
            j0 = k * DUN
            scatters = [
                pltpu.async_copy(ones_v, accd.at[dst_c.at[j0 + b, 0]],
                                 ssem, add=True)
                for b in range(DUN)
            ]
            for sc in scatters:
                sc.wait()
            return carry2

        lax.fori_loop(0, DNI, step, 0)
        return carry

    lax.fori_loop(0, DNG, group, 0)
    plsc.subcore_barrier()
    pltpu.sync_copy(accd.at[pl.ds(r0, STRIPE), :], out_deg.at[c, pl.ds(r0, STRIPE), :])

    @pl.when(s == NS - 1)
    def _():
        pltpu.sync_copy(accd.at[pl.ds(TAIL0, TAILN), :],
                        out_deg.at[c, pl.ds(TAIL0, TAILN), :])


_sc_agg = pl.kernel(
    _sc_agg_body,
    out_type=jax.ShapeDtypeStruct((NC, N, H), jnp.float32),
    mesh=_MESH,
    scratch_types=(
        [pltpu.VMEM((SB, SROWS, 1, CH), jnp.int32)]
        + [pltpu.VMEM((GB, 1, CH), jnp.int32)]
        + [pltpu.VMEM((NB, CH, H), jnp.float32)]
        + [pltpu.VMEM_SHARED((N, H), jnp.float32)]
        + [pltpu.SemaphoreType.DMA] * (NB + 1)
    ),
)

_sc_deg = pl.kernel(
    _sc_deg_body,
    out_type=jax.ShapeDtypeStruct((NC, N, H), jnp.float32),
    mesh=_MESH,
    scratch_types=[
        pltpu.VMEM((DGB, 1, DCH), jnp.int32),
        pltpu.VMEM((DCH, H), jnp.float32),
        pltpu.VMEM_SHARED((N, H), jnp.float32),
        pltpu.SemaphoreType.DMA,
    ],
)


def _tc_in_body(x_ref, w_ref, b_ref, o_ref):
    o_ref[...] = jnp.tanh(
        jnp.dot(x_ref[...], w_ref[...], preferred_element_type=jnp.float32)
        + b_ref[...]
    )


def _tc_mid_body(a0_ref, a1_ref, d0_ref, d1_ref, w_ref, b_ref, o_ref):
    agg = a0_ref[0] + a1_ref[0]
    deg = jnp.maximum(d0_ref[0][:, :1] + d1_ref[0][:, :1], 1.0)
    t = jnp.dot(agg, w_ref[...], preferred_element_type=jnp.float32) / deg
    o_ref[...] = jnp.maximum(t + b_ref[...], 0.0)


def _tc_out_body(a0_ref, a1_ref, d0_ref, d1_ref, w_ref, b_ref, wo_ref, bo_ref,
                 o_ref, acc_ref):
    i = pl.program_id(0)

    @pl.when(i == 0)
    def _():
        acc_ref[...] = jnp.zeros_like(acc_ref)

    agg = a0_ref[0] + a1_ref[0]
    deg = jnp.maximum(d0_ref[0][:, :1] + d1_ref[0][:, :1], 1.0)
    t = jnp.dot(agg, w_ref[...], preferred_element_type=jnp.float32) / deg
    h = jnp.maximum(t + b_ref[...], 0.0)
    acc_ref[...] += jnp.sum(h, axis=0, keepdims=True)

    @pl.when(i == pl.num_programs(0) - 1)
    def _():
        y = jnp.sum(acc_ref[...] * wo_ref[...]) + bo_ref[0, 0]
        o_ref[...] = jnp.full((1, H), y, jnp.float32)


_tc_in = pl.pallas_call(
    _tc_in_body,
    grid=(GRID,),
    in_specs=[
        pl.BlockSpec((BLK, H), lambda i: (i, 0)),
        pl.BlockSpec((H, H), lambda i: (0, 0)),
        pl.BlockSpec((1, H), lambda i: (0, 0)),
    ],
    out_specs=pl.BlockSpec((BLK, H), lambda i: (i, 0)),
    out_shape=jax.ShapeDtypeStruct((N, H), jnp.float32),
)

_tc_mid = pl.pallas_call(
    _tc_mid_body,
    grid=(GRID,),
    in_specs=[
        pl.BlockSpec((1, BLK, H), lambda i: (0, i, 0)),
        pl.BlockSpec((1, BLK, H), lambda i: (1, i, 0)),
        pl.BlockSpec((1, BLK, H), lambda i: (0, i, 0)),
        pl.BlockSpec((1, BLK, H), lambda i: (1, i, 0)),
        pl.BlockSpec((H, H), lambda i: (0, 0)),
        pl.BlockSpec((1, H), lambda i: (0, 0)),
    ],
    out_specs=pl.BlockSpec((BLK, H), lambda i: (i, 0)),
    out_shape=jax.ShapeDtypeStruct((N, H), jnp.float32),
)

_tc_out = pl.pallas_call(
    _tc_out_body,
    grid=(GRID,),
    in_specs=[
        pl.BlockSpec((1, BLK, H), lambda i: (0, i, 0)),
        pl.BlockSpec((1, BLK, H), lambda i: (1, i, 0)),
        pl.BlockSpec((1, BLK, H), lambda i: (0, i, 0)),
        pl.BlockSpec((1, BLK, H), lambda i: (1, i, 0)),
        pl.BlockSpec((H, H), lambda i: (0, 0)),
        pl.BlockSpec((1, H), lambda i: (0, 0)),
        pl.BlockSpec((1, H), lambda i: (0, 0)),
        pl.BlockSpec((1, H), lambda i: (0, 0)),
    ],
    out_specs=pl.BlockSpec((1, H), lambda i: (0, 0)),
    out_shape=jax.ShapeDtypeStruct((1, H), jnp.float32),
    scratch_shapes=[pltpu.VMEM((1, H), jnp.float32)],
)


def kernel(x, edge_index, W_in, b_in, W1, b1, W2, b2, W_out, b_out):
    src = jnp.pad(edge_index[0].astype(jnp.int32).reshape(NW * NCHUNK, 1, CH),
                  ((0, IPAD), (0, 0), (0, 0)))
    dst = edge_index[1].astype(jnp.int32).reshape(NW * NCHUNK, 1, CH)
    zeros = jnp.zeros((N, H), jnp.float32)
    ones = jnp.ones((DCH, H), jnp.float32)

    deg = _sc_deg(dst, zeros, ones)
    h0 = _tc_in(x, W_in, b_in.reshape(1, H))
    agg1 = _sc_agg(h0, src, dst, zeros)
    h1 = _tc_mid(agg1, agg1, deg, deg, W1, b1.reshape(1, H))
    agg2 = _sc_agg(h1, src, dst, zeros)
    y = _tc_out(agg2, agg2, deg, deg, W2, b2.reshape(1, H),
                W_out.reshape(1, H),
                jnp.broadcast_to(b_out.reshape(1, 1), (1, H)))
    return y[0, 0]


# speedup vs baseline: 1.1137x; 1.1137x over previous
import jax
import jax.numpy as jnp
from jax import lax
from jax.experimental import pallas as pl
from jax.experimental.pallas import tpu as pltpu
from jax.experimental.pallas import tpu_sc as plsc

N = 10000
E = 320000
H = 128
NC = 2
NS = 16
NW = NC * NS
CH = 40
NCHUNK = E // (NW * CH)
NB = 5
GB = 10
NG = NCHUNK // GB
NBODY = GB // NB
SB = 3
SROWS = GB + NB
IPAD = 16
DCH = 40
DNCHUNK = E // (NW * DCH)
DUN = 5
DGB = 25
DNG = DNCHUNK // DGB
DNI = DGB // DUN
STRIPE = 624
TAIL0 = NS * STRIPE
TAILN = N - TAIL0
BLK = 1000
GRID = N // BLK

_MESH = plsc.VectorSubcoreMesh(
    core_axis_name="c", subcore_axis_name="s", num_cores=NC, num_subcores=NS
)


def _sc_agg_body(h_hbm, src_hbm, dst_hbm, zeros_hbm,
                 out_agg,
                 src_bk, dst_bk, rows_v, acc,
                 g0, g1, g2, g3, g4, ssem, isem):
    c = lax.axis_index("c")
    s = lax.axis_index("s")
    w = c * NS + s
    r0 = s * STRIPE
    pltpu.sync_copy(zeros_hbm.at[pl.ds(r0, STRIPE), :], acc.at[pl.ds(r0, STRIPE), :])

    @pl.when(s == NS - 1)
    def _():
        pltpu.sync_copy(zeros_hbm.at[pl.ds(TAIL0, TAILN), :],
                        acc.at[pl.ds(TAIL0, TAILN), :])

    plsc.subcore_barrier()

    rows = [rows_v.at[k] for k in range(NB)]
    gsem = [g0, g1, g2, g3, g4]

    def group(g, carry):
        p = lax.rem(g, SB)
        p2 = lax.rem(g, 2)
        pn = lax.rem(g + 1, SB)
        pn2 = lax.rem(g + 1, 2)
        base = w * NCHUNK + g * GB

        @pl.when(g > 0)
        def _():
            pltpu.make_async_copy(
                src_hbm.at[pl.ds(base, SROWS)], src_bk.at[p], isem).wait()
            pltpu.make_async_copy(
                dst_hbm.at[pl.ds(base, GB)], dst_bk.at[p2], isem).wait()

        pltpu.async_copy(src_hbm.at[pl.ds(base + GB, SROWS)], src_bk.at[pn],
                         isem)
        pltpu.async_copy(dst_hbm.at[pl.ds(base + GB, GB)], dst_bk.at[pn2],
                         isem)

        def body(i, carry2):
            t0 = NB * i
            for k in range(NB):
                lc = t0 + k
                pltpu.make_async_copy(
                    h_hbm.at[src_bk.at[p, lc, 0]], rows[k], gsem[k]).wait()
                pltpu.async_copy(
                    rows[k], acc.at[dst_bk.at[p2, lc, 0]], ssem,
                    add=True).wait()
                pltpu.async_copy(h_hbm.at[src_bk.at[p, lc + NB, 0]], rows[k],
                                 gsem[k])
            return carry2

        lax.fori_loop(0, NBODY, body, 0)
        return carry

    pltpu.sync_copy(src_hbm.at[pl.ds(w * NCHUNK, SROWS)], src_bk.at[0])
    pltpu.sync_copy(dst_hbm.at[pl.ds(w * NCHUNK, GB)], dst_bk.at[0])
    for k in range(NB):
        pltpu.async_copy(h_hbm.at[src_bk.at[0, k, 0]], rows[k], gsem[k])
    lax.fori_loop(0, NG, group, 0)
    for k in range(NB):
        pltpu.make_async_copy(h_hbm.at[pl.ds(0, CH)], rows[k], gsem[k]).wait()
    pltpu.make_async_copy(src_hbm.at[pl.ds(0, SROWS)], src_bk.at[0],
                          isem).wait()
    pltpu.make_async_copy(dst_hbm.at[pl.ds(0, GB)], dst_bk.at[0], isem).wait()
    plsc.subcore_barrier()
    pltpu.sync_copy(acc.at[pl.ds(r0, STRIPE), :], out_agg.at[c, pl.ds(r0, STRIPE), :])

    @pl.when(s == NS - 1)
    def _():
        pltpu.sync_copy(acc.at[pl.ds(TAIL0, TAILN), :],
                        out_agg.at[c, pl.ds(TAIL0, TAILN), :])


def _sc_deg_body(dst_hbm, zeros_hbm, ones_hbm,
                 out_deg,
                 dst_c, ones_v, accd, ssem):
    c = lax.axis_index("c")
    s = lax.axis_index("s")
    w = c * NS + s
    r0 = s * STRIPE
    pltpu.sync_copy(zeros_hbm.at[pl.ds(r0, STRIPE), :], accd.at[pl.ds(r0, STRIPE), :])

    @pl.when(s == NS - 1)
    def _():
        pltpu.sync_copy(zeros_hbm.at[pl.ds(TAIL0, TAILN), :],
                        accd.at[pl.ds(TAIL0, TAILN), :])

    pltpu.sync_copy(ones_hbm, ones_v)
    plsc.subcore_barrier()

    def group(g, carry):
        base = w * DNCHUNK + g * DGB
        pltpu.sync_copy(dst_hbm.at[pl.ds(base, DGB)], dst_c)

        def step(k, carry2):
            j0 = k * DUN
            scatters = [
                pltpu.async_copy(ones_v, accd.at[dst_c.at[j0 + b, 0]],
                                 ssem, add=True)
                for b in range(DUN)
            ]
            for sc in scatters:
                sc.wait()
            return carry2

        lax.fori_loop(0, DNI, step, 0)
        return carry

    lax.fori_loop(0, DNG, group, 0)
    plsc.subcore_barrier()
    pltpu.sync_copy(accd.at[pl.ds(r0, STRIPE), :], out_deg.at[c, pl.ds(r0, STRIPE), :])

    @pl.when(s == NS - 1)
    def _():
        pltpu.sync_copy(accd.at[pl.ds(TAIL0, TAILN), :],
                        out_deg.at[c, pl.ds(TAIL0, TAILN), :])


_sc_agg = pl.kernel(
    _sc_agg_body,
    out_type=jax.ShapeDtypeStruct((NC, N, H), jnp.float32),
    mesh=_MESH,
    scratch_types=(
        [pltpu.VMEM((SB, SROWS, 1, CH), jnp.int32)]
        + [pltpu.VMEM((2, GB, 1, CH), jnp.int32)]
        + [pltpu.VMEM((NB, CH, H), jnp.float32)]
        + [pltpu.VMEM_SHARED((N, H), jnp.float32)]
        + [pltpu.SemaphoreType.DMA] * (NB + 2)
    ),
)

_sc_deg = pl.kernel(
    _sc_deg_body,
    out_type=jax.ShapeDtypeStruct((NC, N, H), jnp.float32),
    mesh=_MESH,
    scratch_types=[
        pltpu.VMEM((DGB, 1, DCH), jnp.int32),
        pltpu.VMEM((DCH, H), jnp.float32),
        pltpu.VMEM_SHARED((N, H), jnp.float32),
        pltpu.SemaphoreType.DMA,
    ],
)


def _tc_in_body(x_ref, w_ref, b_ref, o_ref):
    o_ref[...] = jnp.tanh(
        jnp.dot(x_ref[...], w_ref[...], preferred_element_type=jnp.float32)
        + b_ref[...]
    )


def _tc_mid_body(a0_ref, a1_ref, d0_ref, d1_ref, w_ref, b_ref, o_ref):
    agg = a0_ref[0] + a1_ref[0]
    deg = jnp.maximum(d0_ref[0][:, :1] + d1_ref[0][:, :1], 1.0)
    t = jnp.dot(agg, w_ref[...], preferred_element_type=jnp.float32) / deg
    o_ref[...] = jnp.maximum(t + b_ref[...], 0.0)


def _tc_out_body(a0_ref, a1_ref, d0_ref, d1_ref, w_ref, b_ref, wo_ref, bo_ref,
                 o_ref, acc_ref):
    i = pl.program_id(0)

    @pl.when(i == 0)
    def _():
        acc_ref[...] = jnp.zeros_like(acc_ref)

    agg = a0_ref[0] + a1_ref[0]
    deg = jnp.maximum(d0_ref[0][:, :1] + d1_ref[0][:, :1], 1.0)
    t = jnp.dot(agg, w_ref[...], preferred_element_type=jnp.float32) / deg
    h = jnp.maximum(t + b_ref[...], 0.0)
    acc_ref[...] += jnp.sum(h, axis=0, keepdims=True)

    @pl.when(i == pl.num_programs(0) - 1)
    def _():
        y = jnp.sum(acc_ref[...] * wo_ref[...]) + bo_ref[0, 0]
        o_ref[...] = jnp.full((1, H), y, jnp.float32)


_tc_in = pl.pallas_call(
    _tc_in_body,
    grid=(GRID,),
    in_specs=[
        pl.BlockSpec((BLK, H), lambda i: (i, 0)),
        pl.BlockSpec((H, H), lambda i: (0, 0)),
        pl.BlockSpec((1, H), lambda i: (0, 0)),
    ],
    out_specs=pl.BlockSpec((BLK, H), lambda i: (i, 0)),
    out_shape=jax.ShapeDtypeStruct((N, H), jnp.float32),
)

_tc_mid = pl.pallas_call(
    _tc_mid_body,
    grid=(GRID,),
    in_specs=[
        pl.BlockSpec((1, BLK, H), lambda i: (0, i, 0)),
        pl.BlockSpec((1, BLK, H), lambda i: (1, i, 0)),
        pl.BlockSpec((1, BLK, H), lambda i: (0, i, 0)),
        pl.BlockSpec((1, BLK, H), lambda i: (1, i, 0)),
        pl.BlockSpec((H, H), lambda i: (0, 0)),
        pl.BlockSpec((1, H), lambda i: (0, 0)),
    ],
    out_specs=pl.BlockSpec((BLK, H), lambda i: (i, 0)),
    out_shape=jax.ShapeDtypeStruct((N, H), jnp.float32),
)

_tc_out = pl.pallas_call(
    _tc_out_body,
    grid=(GRID,),
    in_specs=[
        pl.BlockSpec((1, BLK, H), lambda i: (0, i, 0)),
        pl.BlockSpec((1, BLK, H), lambda i: (1, i, 0)),
        pl.BlockSpec((1, BLK, H), lambda i: (0, i, 0)),
        pl.BlockSpec((1, BLK, H), lambda i: (1, i, 0)),
        pl.BlockSpec((H, H), lambda i: (0, 0)),
        pl.BlockSpec((1, H), lambda i: (0, 0)),
        pl.BlockSpec((1, H), lambda i: (0, 0)),
        pl.BlockSpec((1, H), lambda i: (0, 0)),
    ],
    out_specs=pl.BlockSpec((1, H), lambda i: (0, 0)),
    out_shape=jax.ShapeDtypeStruct((1, H), jnp.float32),
    scratch_shapes=[pltpu.VMEM((1, H), jnp.float32)],
)


def kernel(x, edge_index, W_in, b_in, W1, b1, W2, b2, W_out, b_out):
    pad3 = ((0, IPAD), (0, 0), (0, 0))
    src = jnp.pad(edge_index[0].astype(jnp.int32).reshape(NW * NCHUNK, 1, CH),
                  pad3)
    dst = jnp.pad(edge_index[1].astype(jnp.int32).reshape(NW * NCHUNK, 1, CH),
                  pad3)
    zeros = jnp.zeros((N, H), jnp.float32)
    ones = jnp.ones((DCH, H), jnp.float32)

    deg = _sc_deg(dst, zeros, ones)
    h0 = _tc_in(x, W_in, b_in.reshape(1, H))
    agg1 = _sc_agg(h0, src, dst, zeros)
    h1 = _tc_mid(agg1, agg1, deg, deg, W1, b1.reshape(1, H))
    agg2 = _sc_agg(h1, src, dst, zeros)
    y = _tc_out(agg2, agg2, deg, deg, W2, b2.reshape(1, H),
                W_out.reshape(1, H),
                jnp.broadcast_to(b_out.reshape(1, 1), (1, H)))
    return y[0, 0]
